# baseline (device time: 19358 ns/iter reference)
import jax
import jax.numpy as jnp
from jax import lax
from jax.experimental import pallas as pl
from jax.experimental.pallas import tpu as pltpu


def kernel(partial, resid, gamma):
    _, m, d = partial.shape
    gamma2 = gamma.reshape(1, d)

    def body(p_ref, r_ref, g_ref, o_ref, comm_ref, send_sem, recv_sem):
        my_x = lax.axis_index("x")
        my_y = lax.axis_index("y")
        nbr = (my_x, 1 - my_y)

        barrier_sem = pltpu.get_barrier_semaphore()
        pl.semaphore_signal(
            barrier_sem, inc=1, device_id=nbr,
            device_id_type=pl.DeviceIdType.MESH,
        )
        pl.semaphore_wait(barrier_sem, 1)

        rdma = pltpu.make_async_remote_copy(
            src_ref=p_ref.at[0],
            dst_ref=comm_ref,
            send_sem=send_sem,
            recv_sem=recv_sem,
            device_id=nbr,
            device_id_type=pl.DeviceIdType.MESH,
        )
        rdma.start()
        rdma.wait()

        y = p_ref[0] + comm_ref[...] + r_ref[...]
        rms = jnp.sqrt(jnp.mean(y * y, axis=-1, keepdims=True) + 1e-6)
        o_ref[...] = y / rms * g_ref[...]

    return pl.pallas_call(
        body,
        out_shape=jax.ShapeDtypeStruct((m, d), jnp.float32),
        in_specs=[
            pl.BlockSpec(memory_space=pltpu.VMEM),
            pl.BlockSpec(memory_space=pltpu.VMEM),
            pl.BlockSpec(memory_space=pltpu.VMEM),
        ],
        out_specs=pl.BlockSpec(memory_space=pltpu.VMEM),
        scratch_shapes=[
            pltpu.VMEM((m, d), jnp.float32),
            pltpu.SemaphoreType.DMA,
            pltpu.SemaphoreType.DMA,
        ],
        compiler_params=pltpu.CompilerParams(collective_id=0),
    )(partial, resid, gamma2)


# device time: 17464 ns/iter; 1.1085x vs baseline; 1.1085x over previous
import jax
import jax.numpy as jnp
from jax import lax
from jax.experimental import pallas as pl
from jax.experimental.pallas import tpu as pltpu

K = 4


def kernel(partial, resid, gamma):
    _, m, d = partial.shape
    half = m // 2
    cm = half // K
    gamma2 = gamma.reshape(1, d)

    def body(
        p_ref, r_ref, g_ref, o_ref, comm_ref,
        y_send_sems, y_recv_sems, x_send_sems, x_recv_sems,
    ):
        my_x = lax.axis_index("x")
        my_y = lax.axis_index("y")
        y_nbr = (my_x, 1 - my_y)
        x_nbr = (1 - my_x, my_y)
        h_own = (my_x + my_y) % 2
        h_nbr = 1 - h_own

        barrier_sem = pltpu.get_barrier_semaphore()
        for nbr in (y_nbr, x_nbr):
            pl.semaphore_signal(
                barrier_sem, inc=1, device_id=nbr,
                device_id_type=pl.DeviceIdType.MESH,
            )
        pl.semaphore_wait(barrier_sem, 2)

        y_rdmas = []
        for k in range(K):
            rows = pl.ds(h_nbr * half + k * cm, cm)
            rd = pltpu.make_async_remote_copy(
                src_ref=p_ref.at[0, rows, :],
                dst_ref=comm_ref.at[k],
                send_sem=y_send_sems.at[k],
                recv_sem=y_recv_sems.at[k],
                device_id=y_nbr,
                device_id_type=pl.DeviceIdType.MESH,
            )
            rd.start()
            y_rdmas.append(rd)

        x_rdmas = []
        for k in range(K):
            y_rdmas[k].wait_recv()
            rows = pl.ds(h_own * half + k * cm, cm)
            y = p_ref[0, rows, :] + comm_ref[k] + r_ref[rows, :]
            rms = jnp.sqrt(jnp.mean(y * y, axis=-1, keepdims=True) + 1e-6)
            o_ref[rows, :] = y / rms * g_ref[...]
            rd = pltpu.make_async_remote_copy(
                src_ref=o_ref.at[rows, :],
                dst_ref=o_ref.at[rows, :],
                send_sem=x_send_sems.at[k],
                recv_sem=x_recv_sems.at[k],
                device_id=x_nbr,
                device_id_type=pl.DeviceIdType.MESH,
            )
            rd.start()
            x_rdmas.append(rd)

        for k in range(K):
            x_rdmas[k].wait_recv()
            y_rdmas[k].wait_send()
            x_rdmas[k].wait_send()

    return pl.pallas_call(
        body,
        out_shape=jax.ShapeDtypeStruct((m, d), jnp.float32),
        in_specs=[
            pl.BlockSpec(memory_space=pltpu.VMEM),
            pl.BlockSpec(memory_space=pltpu.VMEM),
            pl.BlockSpec(memory_space=pltpu.VMEM),
        ],
        out_specs=pl.BlockSpec(memory_space=pltpu.VMEM),
        scratch_shapes=[
            pltpu.VMEM((K, cm, d), jnp.float32),
            pltpu.SemaphoreType.DMA((K,)),
            pltpu.SemaphoreType.DMA((K,)),
            pltpu.SemaphoreType.DMA((K,)),
            pltpu.SemaphoreType.DMA((K,)),
        ],
        compiler_params=pltpu.CompilerParams(collective_id=0),
    )(partial, resid, gamma2)


# device time: 16888 ns/iter; 1.1463x vs baseline; 1.0341x over previous
import functools

import jax
import jax.numpy as jnp
from jax import lax
from jax.experimental import pallas as pl
from jax.experimental.pallas import tpu as pltpu

K = 8


def kernel(partial, resid, gamma):
    _, m, d = partial.shape
    half = m // 2
    cm = half // K

    def body(
        p_ref, r_ref, g_ref, o_ref, comm_ref,
        y_send_sems, y_recv_sems, x_send_sems, x_recv_sems,
    ):
        my_x = lax.axis_index("x")
        my_y = lax.axis_index("y")
        y_nbr = (my_x, 1 - my_y)
        x_nbr = (1 - my_x, my_y)
        h_own = (my_x + my_y) % 2
        h_nbr = 1 - h_own

        barrier_sem = pltpu.get_barrier_semaphore()
        for nbr in (y_nbr, x_nbr):
            pl.semaphore_signal(
                barrier_sem, inc=1, device_id=nbr,
                device_id_type=pl.DeviceIdType.MESH,
            )
        pl.semaphore_wait(barrier_sem, 2)

        y_rdmas = []
        for k in range(K):
            rows = pl.ds(h_nbr * half + k * cm, cm)
            rd = pltpu.make_async_remote_copy(
                src_ref=p_ref.at[0, rows, :],
                dst_ref=comm_ref.at[k],
                send_sem=y_send_sems.at[k],
                recv_sem=y_recv_sems.at[k],
                device_id=y_nbr,
                device_id_type=pl.DeviceIdType.MESH,
            )
            rd.start()
            y_rdmas.append(rd)

        x_rdmas = []
        for k in range(K):
            y_rdmas[k].wait_recv()
            rows = pl.ds(h_own * half + k * cm, cm)
            y = p_ref[0, rows, :] + comm_ref[k] + r_ref[rows, :]
            ms = jnp.mean(y * y, axis=-1, keepdims=True)
            o_ref[rows, :] = y * lax.rsqrt(ms + 1e-6) * g_ref[...]
            rd = pltpu.make_async_remote_copy(
                src_ref=o_ref.at[rows, :],
                dst_ref=o_ref.at[rows, :],
                send_sem=x_send_sems.at[k],
                recv_sem=x_recv_sems.at[k],
                device_id=x_nbr,
                device_id_type=pl.DeviceIdType.MESH,
            )
            rd.start()
            x_rdmas.append(rd)

        for k in range(K):
            x_rdmas[k].wait_recv()
            y_rdmas[k].wait_send()
            x_rdmas[k].wait_send()

    return pl.pallas_call(
        body,
        out_shape=jax.ShapeDtypeStruct((m, d), jnp.float32),
        in_specs=[
            pl.BlockSpec(memory_space=pltpu.VMEM),
            pl.BlockSpec(memory_space=pltpu.VMEM),
            pl.BlockSpec(memory_space=pltpu.VMEM),
        ],
        out_specs=pl.BlockSpec(memory_space=pltpu.VMEM),
        scratch_shapes=[
            pltpu.VMEM((K, cm, d), jnp.float32),
            pltpu.SemaphoreType.DMA((K,)),
            pltpu.SemaphoreType.DMA((K,)),
            pltpu.SemaphoreType.DMA((K,)),
            pltpu.SemaphoreType.DMA((K,)),
        ],
        compiler_params=pltpu.CompilerParams(collective_id=0),
    )(partial, resid, gamma)


# device time: 16851 ns/iter; 1.1488x vs baseline; 1.0022x over previous
import jax
import jax.numpy as jnp
from jax import lax
from jax.experimental import pallas as pl
from jax.experimental.pallas import tpu as pltpu

K = 16


def kernel(partial, resid, gamma):
    _, m, d = partial.shape
    half = m // 2
    cm = half // K
    partial2 = partial.reshape(m, d)

    def body(
        p_ref, r_ref, g_ref, o_ref, comm_ref, pre_ref,
        y_send_sems, y_recv_sems, x_send_sems, x_recv_sems,
    ):
        my_x = lax.axis_index("x")
        my_y = lax.axis_index("y")
        y_nbr = (my_x, 1 - my_y)
        x_nbr = (1 - my_x, my_y)
        h_own = (my_x + my_y) % 2
        h_nbr = 1 - h_own

        barrier_sem = pltpu.get_barrier_semaphore()
        for nbr in (y_nbr, x_nbr):
            pl.semaphore_signal(
                barrier_sem, inc=1, device_id=nbr,
                device_id_type=pl.DeviceIdType.MESH,
            )
        pl.semaphore_wait(barrier_sem, 2)

        y_rdmas = []
        for k in range(K):
            rows = pl.ds(h_nbr * half + k * cm, cm)
            rd = pltpu.make_async_remote_copy(
                src_ref=p_ref.at[rows, :],
                dst_ref=comm_ref.at[k],
                send_sem=y_send_sems.at[k],
                recv_sem=y_recv_sems.at[k],
                device_id=y_nbr,
                device_id_type=pl.DeviceIdType.MESH,
            )
            rd.start()
            y_rdmas.append(rd)

        own = pl.ds(h_own * half, half)
        pre_ref[...] = p_ref[own, :] + r_ref[own, :]
        g = g_ref[...]

        x_rdmas = []
        for k in range(K):
            y_rdmas[k].wait_recv()
            y = pre_ref[k * cm:(k + 1) * cm, :] + comm_ref[k]
            ms = jnp.mean(y * y, axis=-1, keepdims=True)
            rows = pl.ds(h_own * half + k * cm, cm)
            o_ref[rows, :] = y * lax.rsqrt(ms + 1e-6) * g
            rd = pltpu.make_async_remote_copy(
                src_ref=o_ref.at[rows, :],
                dst_ref=o_ref.at[rows, :],
                send_sem=x_send_sems.at[k],
                recv_sem=x_recv_sems.at[k],
                device_id=x_nbr,
                device_id_type=pl.DeviceIdType.MESH,
            )
            rd.start()
            x_rdmas.append(rd)

        for k in range(K):
            x_rdmas[k].wait_recv()
            y_rdmas[k].wait_send()
            x_rdmas[k].wait_send()

    return pl.pallas_call(
        body,
        out_shape=jax.ShapeDtypeStruct((m, d), jnp.float32),
        in_specs=[
            pl.BlockSpec(memory_space=pltpu.VMEM),
            pl.BlockSpec(memory_space=pltpu.VMEM),
            pl.BlockSpec(memory_space=pltpu.VMEM),
        ],
        out_specs=pl.BlockSpec(memory_space=pltpu.VMEM),
        scratch_shapes=[
            pltpu.VMEM((K, cm, d), jnp.float32),
            pltpu.VMEM((half, d), jnp.float32),
            pltpu.SemaphoreType.DMA((K,)),
            pltpu.SemaphoreType.DMA((K,)),
            pltpu.SemaphoreType.DMA((K,)),
            pltpu.SemaphoreType.DMA((K,)),
        ],
        compiler_params=pltpu.CompilerParams(collective_id=0),
    )(partial2, resid, gamma)


# device time: 16342 ns/iter; 1.1846x vs baseline; 1.0311x over previous
import jax
import jax.numpy as jnp
from jax import lax
from jax.experimental import pallas as pl
from jax.experimental.pallas import tpu as pltpu

CM = 16
KO = 16
KE = 3
KF = KO - KE
KY = KO + KE


def kernel(partial, resid, gamma):
    _, m, d = partial.shape
    half = m // 2
    fwd_rows = KF * CM
    partial2 = partial.reshape(m, d)

    def body(
        p_ref, r_ref, g_ref, o_ref, comm_ref, pre_ref,
        y_send_sems, y_recv_sems, x_send_sems, x_recv_sems,
    ):
        my_x = lax.axis_index("x")
        my_y = lax.axis_index("y")
        y_nbr = (my_x, 1 - my_y)
        x_nbr = (1 - my_x, my_y)
        h_own = (my_x + my_y) % 2
        h_nbr = 1 - h_own

        barrier_sem = pltpu.get_barrier_semaphore()
        for nbr in (y_nbr, x_nbr):
            pl.semaphore_signal(
                barrier_sem, inc=1, device_id=nbr,
                device_id_type=pl.DeviceIdType.MESH,
            )
        pl.semaphore_wait(barrier_sem, 2)

        y_rdmas = []
        for k in range(KY):
            if k < KO:
                src_rows = pl.ds(h_nbr * half + k * CM, CM)
            else:
                src_rows = pl.ds(
                    h_own * half + fwd_rows + (k - KO) * CM, CM
                )
            rd = pltpu.make_async_remote_copy(
                src_ref=p_ref.at[src_rows, :],
                dst_ref=comm_ref.at[k],
                send_sem=y_send_sems.at[k],
                recv_sem=y_recv_sems.at[k],
                device_id=y_nbr,
                device_id_type=pl.DeviceIdType.MESH,
            )
            rd.start()
            y_rdmas.append(rd)

        own = pl.ds(h_own * half, half)
        pre_ref[...] = p_ref[own, :] + r_ref[own, :]
        g = g_ref[...]

        x_rdmas = []
        for k in range(KO):
            y_rdmas[k].wait_recv()
            y = pre_ref[k * CM:(k + 1) * CM, :] + comm_ref[k]
            ms = jnp.mean(y * y, axis=-1, keepdims=True)
            rows = pl.ds(h_own * half + k * CM, CM)
            o_ref[rows, :] = y * lax.rsqrt(ms + 1e-6) * g
            if k < KF:
                rd = pltpu.make_async_remote_copy(
                    src_ref=o_ref.at[rows, :],
                    dst_ref=o_ref.at[rows, :],
                    send_sem=x_send_sems.at[k],
                    recv_sem=x_recv_sems.at[k],
                    device_id=x_nbr,
                    device_id_type=pl.DeviceIdType.MESH,
                )
                rd.start()
                x_rdmas.append(rd)

        for j in range(KE):
            y_rdmas[KO + j].wait_recv()
            rows = pl.ds(h_nbr * half + fwd_rows + j * CM, CM)
            y = p_ref[rows, :] + r_ref[rows, :] + comm_ref[KO + j]
            ms = jnp.mean(y * y, axis=-1, keepdims=True)
            o_ref[rows, :] = y * lax.rsqrt(ms + 1e-6) * g

        for k in range(KF):
            x_rdmas[k].wait_recv()
            x_rdmas[k].wait_send()
        for k in range(KY):
            y_rdmas[k].wait_send()

    return pl.pallas_call(
        body,
        out_shape=jax.ShapeDtypeStruct((m, d), jnp.float32),
        in_specs=[
            pl.BlockSpec(memory_space=pltpu.VMEM),
            pl.BlockSpec(memory_space=pltpu.VMEM),
            pl.BlockSpec(memory_space=pltpu.VMEM),
        ],
        out_specs=pl.BlockSpec(memory_space=pltpu.VMEM),
        scratch_shapes=[
            pltpu.VMEM((KY, CM, d), jnp.float32),
            pltpu.VMEM((half, d), jnp.float32),
            pltpu.SemaphoreType.DMA((KY,)),
            pltpu.SemaphoreType.DMA((KY,)),
            pltpu.SemaphoreType.DMA((KF,)),
            pltpu.SemaphoreType.DMA((KF,)),
        ],
        compiler_params=pltpu.CompilerParams(collective_id=0),
    )(partial2, resid, gamma)
